# baseline (speedup 1.0000x reference)
import functools

import jax
import jax.numpy as jnp
from jax import lax
from jax.experimental import pallas as pl
from jax.experimental.pallas import tpu as pltpu
from jax.experimental.pallas import tpu_sc as plsc

BATCH = 16384
EMBED_DIM = 64
LANES = 16
VREGS_PER_ROW = EMBED_DIM // LANES

NUM_CORES = 2
NUM_SUBCORES = 16
NUM_WORKERS = NUM_CORES * NUM_SUBCORES
B_PER_W = BATCH // NUM_WORKERS
IDX_CHUNK = 128
N_CHUNKS = B_PER_W // IDX_CHUNK
GROUPS = B_PER_W // LANES
GROUPS_PER_CHUNK = IDX_CHUNK // LANES


def _make_sc_kernel():
    mesh = plsc.VectorSubcoreMesh(core_axis_name="c", subcore_axis_name="s")

    @functools.partial(
        pl.kernel,
        mesh=mesh,
        compiler_params=pltpu.CompilerParams(
            needs_layout_passes=False, use_tc_tiling_on_sc=False),
        out_type=jax.ShapeDtypeStruct((BATCH,), jnp.float32),
        scratch_types=[
            pltpu.VMEM((N_CHUNKS, IDX_CHUNK), jnp.int32),
            pltpu.VMEM((N_CHUNKS, IDX_CHUNK), jnp.int32),
            pltpu.VMEM((N_CHUNKS, IDX_CHUNK, EMBED_DIM), jnp.float32),
            pltpu.VMEM((N_CHUNKS, IDX_CHUNK, EMBED_DIM), jnp.float32),
            pltpu.VMEM((B_PER_W,), jnp.float32),
            pltpu.SemaphoreType.DMA,
        ],
    )
    def sc_kernel(user_hbm, item_hbm, utab_hbm, itab_hbm, out_hbm,
                  uidx, iidx, urows, irows, outv, sem):
        wid = lax.axis_index("s") * NUM_CORES + lax.axis_index("c")
        base = wid * B_PER_W

        for ch in range(N_CHUNKS):
            pltpu.sync_copy(user_hbm.at[pl.ds(base + ch * IDX_CHUNK, IDX_CHUNK)],
                            uidx.at[ch])
            pltpu.sync_copy(item_hbm.at[pl.ds(base + ch * IDX_CHUNK, IDX_CHUNK)],
                            iidx.at[ch])

        copies = []
        for ch in range(N_CHUNKS):
            copies.append(pltpu.async_copy(utab_hbm.at[uidx.at[ch]],
                                           urows.at[ch], sem))
            copies.append(pltpu.async_copy(itab_hbm.at[iidx.at[ch]],
                                           irows.at[ch], sem))
        for c in copies:
            c.wait()

        lane_iota = lax.iota(jnp.int32, LANES)

        def group_body(g, carry):
            ch = g // GROUPS_PER_CHUNK
            b0 = (g % GROUPS_PER_CHUNK) * LANES
            acc = jnp.zeros((LANES,), jnp.float32)
            for bs in range(LANES):
                b = b0 + bs
                p = None
                for j in range(VREGS_PER_ROW):
                    u = urows[ch, b, pl.ds(j * LANES, LANES)]
                    v = irows[ch, b, pl.ds(j * LANES, LANES)]
                    p = u * v if p is None else p + u * v
                acc = jnp.where(lane_iota == bs, jnp.sum(p), acc)
            outv[pl.ds(g * LANES, LANES)] = 1.0 / (1.0 + jnp.exp(-acc))
            return carry

        lax.fori_loop(0, GROUPS, group_body, 0)

        pltpu.sync_copy(outv, out_hbm.at[pl.ds(base, B_PER_W)])

    return sc_kernel


_SC_KERNEL = _make_sc_kernel()


def kernel(user, item, user_table, item_table):
    return _SC_KERNEL(user, item, user_table, item_table)
